# baseline (speedup 1.0000x reference)
import jax
import jax.numpy as jnp
from jax import lax
from jax.experimental import pallas as pl
from jax.experimental.pallas import tpu as pltpu
from jax.experimental.pallas import tpu_sc as plsc

VOCAB = 49408
HIDDEN = 768
MAX_POS = 77
BATCH = 1024
SEQ = 77

NC = 2
NS = 16
NW = NC * NS
ROWS = BATCH * SEQ
RPW = ROWS // NW
CH = 16
CPW = RPW // CH
NBUF = 7
LEAD = 4
VPR = HIDDEN // 16
NPOS = 4


def _body(idx_hbm, token_hbm, pos_hbm, out_hbm,
          idx_v, pos_v, buf,
          gsem0, gsem1, gsem2, gsem3, gsem4, gsem5, gsem6,
          wsem0, wsem1, wsem2, wsem3, wsem4, wsem5, wsem6):
  gsems = [gsem0, gsem1, gsem2, gsem3, gsem4, gsem5, gsem6]
  wsems = [wsem0, wsem1, wsem2, wsem3, wsem4, wsem5, wsem6]
  wid = lax.axis_index("s") * NC + lax.axis_index("c")
  rbase = wid * RPW
  s_lo = rbase // BATCH

  pltpu.sync_copy(pos_hbm.at[pl.ds(s_lo * HIDDEN, NPOS * HIDDEN)], pos_v)
  pltpu.sync_copy(idx_hbm.at[pl.ds(rbase, RPW)], idx_v)

  def gather(c, slot):
    idxvec = idx_v[pl.ds(c * CH, CH)]
    return pltpu.make_async_copy(token_hbm.at[idxvec], buf.at[slot],
                                 gsems[slot])

  def wb(c, slot):
    return pltpu.make_async_copy(
        buf.at[slot], out_hbm.at[pl.ds(rbase + c * CH, CH)], wsems[slot])

  for c in range(LEAD):
    gather(c, c).start()

  @pl.loop(0, CPW, step=NBUF)
  def _chunks(c0):
    for b in range(NBUF):
      c = c0 + b
      gather(c, b).wait()

      srel = (rbase + c * CH) // BATCH - s_lo

      @pl.loop(0, VPR, unroll=2)
      def _cols(v):
        pv = pos_v[pl.ds(srel * HIDDEN + v * 16, 16)]
        for r in range(CH):
          plsc.addupdate(buf.at[b, r, pl.ds(v * 16, 16)], pv)

      wb(c, b).start()

      slot_n = (b + LEAD) % NBUF

      @pl.when(c + LEAD < CPW)
      def _():
        @pl.when(c + LEAD - NBUF >= 0)
        def _():
          wb(c + LEAD - NBUF, slot_n).wait()

        gather(c + LEAD, slot_n).start()

  for c in range(CPW - NBUF, CPW):
    wb(c, c % NBUF).wait()


@jax.jit
def _run(ids_t_flat, token_table, pos_flat):
  mesh = plsc.VectorSubcoreMesh(core_axis_name="c", subcore_axis_name="s",
                                num_cores=NC, num_subcores=NS)
  f = pl.kernel(
      _body,
      out_type=jax.ShapeDtypeStruct((ROWS, HIDDEN), jnp.float32),
      mesh=mesh,
      scratch_types=[
          pltpu.VMEM((RPW,), jnp.int32),
          pltpu.VMEM((NPOS * HIDDEN,), jnp.float32),
          pltpu.VMEM((NBUF, CH, HIDDEN), jnp.float32),
          pltpu.SemaphoreType.DMA,
          pltpu.SemaphoreType.DMA,
          pltpu.SemaphoreType.DMA,
          pltpu.SemaphoreType.DMA,
          pltpu.SemaphoreType.DMA,
          pltpu.SemaphoreType.DMA,
          pltpu.SemaphoreType.DMA,
          pltpu.SemaphoreType.DMA,
          pltpu.SemaphoreType.DMA,
          pltpu.SemaphoreType.DMA,
          pltpu.SemaphoreType.DMA,
          pltpu.SemaphoreType.DMA,
          pltpu.SemaphoreType.DMA,
          pltpu.SemaphoreType.DMA,
      ],
  )
  return f(ids_t_flat, token_table, pos_flat)


def kernel(input_ids, position_ids, token_table, pos_table):
  del position_ids
  ids_t_flat = input_ids.astype(jnp.int32).T.reshape(ROWS)
  pos_flat = jnp.concatenate(
      [pos_table.reshape(MAX_POS * HIDDEN),
       jnp.zeros((NPOS * HIDDEN,), jnp.float32)])
  out = _run(ids_t_flat, token_table, pos_flat)
  return out.reshape(SEQ, BATCH, HIDDEN).transpose(1, 0, 2)


# speedup vs baseline: 1.0102x; 1.0102x over previous
import jax
import jax.numpy as jnp
from jax import lax
from jax.experimental import pallas as pl
from jax.experimental.pallas import tpu as pltpu
from jax.experimental.pallas import tpu_sc as plsc

VOCAB = 49408
HIDDEN = 768
MAX_POS = 77
BATCH = 1024
SEQ = 77

NC = 2
NS = 16
NW = NC * NS
ROWS = BATCH * SEQ
RPW = ROWS // NW
CH = 16
CPW = RPW // CH
NBUF = 7
LEAD = 3
VPR = HIDDEN // 16
NPOS = 4


def _body(idx_hbm, token_hbm, pos_hbm, out_hbm,
          idx_v, pos_v, buf,
          gsem0, gsem1, gsem2, gsem3, gsem4, gsem5, gsem6,
          wsem0, wsem1, wsem2, wsem3, wsem4, wsem5, wsem6):
  gsems = [gsem0, gsem1, gsem2, gsem3, gsem4, gsem5, gsem6]
  wsems = [wsem0, wsem1, wsem2, wsem3, wsem4, wsem5, wsem6]
  wid = lax.axis_index("s") * NC + lax.axis_index("c")
  rbase = wid * RPW
  s_lo = rbase // BATCH

  pltpu.sync_copy(pos_hbm.at[pl.ds(s_lo * HIDDEN, NPOS * HIDDEN)], pos_v)
  pltpu.sync_copy(idx_hbm.at[pl.ds(rbase, RPW)], idx_v)

  def gather(c, slot):
    idxvec = idx_v[pl.ds(c * CH, CH)]
    return pltpu.make_async_copy(token_hbm.at[idxvec], buf.at[slot],
                                 gsems[slot])

  def wb(c, slot):
    return pltpu.make_async_copy(
        buf.at[slot], out_hbm.at[pl.ds(rbase + c * CH, CH)], wsems[slot])

  for c in range(LEAD):
    gather(c, c).start()

  @pl.loop(0, CPW, step=NBUF)
  def _chunks(c0):
    for b in range(NBUF):
      c = c0 + b
      gather(c, b).wait()

      srel = (rbase + c * CH) // BATCH - s_lo

      @pl.loop(0, VPR)
      def _cols(v):
        pv = pos_v[pl.ds(srel * HIDDEN + v * 16, 16)]
        for r in range(CH):
          plsc.addupdate(buf.at[b, r, pl.ds(v * 16, 16)], pv)

      wb(c, b).start()

      slot_n = (b + LEAD) % NBUF

      @pl.when(c + LEAD < CPW)
      def _():
        @pl.when(c + LEAD - NBUF >= 0)
        def _():
          wb(c + LEAD - NBUF, slot_n).wait()

        gather(c + LEAD, slot_n).start()

  for c in range(CPW - NBUF, CPW):
    wb(c, c % NBUF).wait()


@jax.jit
def _run(ids_t_flat, token_table, pos_flat):
  mesh = plsc.VectorSubcoreMesh(core_axis_name="c", subcore_axis_name="s",
                                num_cores=NC, num_subcores=NS)
  f = pl.kernel(
      _body,
      out_type=jax.ShapeDtypeStruct((ROWS, HIDDEN), jnp.float32),
      mesh=mesh,
      scratch_types=[
          pltpu.VMEM((RPW,), jnp.int32),
          pltpu.VMEM((NPOS * HIDDEN,), jnp.float32),
          pltpu.VMEM((NBUF, CH, HIDDEN), jnp.float32),
          pltpu.SemaphoreType.DMA,
          pltpu.SemaphoreType.DMA,
          pltpu.SemaphoreType.DMA,
          pltpu.SemaphoreType.DMA,
          pltpu.SemaphoreType.DMA,
          pltpu.SemaphoreType.DMA,
          pltpu.SemaphoreType.DMA,
          pltpu.SemaphoreType.DMA,
          pltpu.SemaphoreType.DMA,
          pltpu.SemaphoreType.DMA,
          pltpu.SemaphoreType.DMA,
          pltpu.SemaphoreType.DMA,
          pltpu.SemaphoreType.DMA,
          pltpu.SemaphoreType.DMA,
      ],
  )
  return f(ids_t_flat, token_table, pos_flat)


def kernel(input_ids, position_ids, token_table, pos_table):
  del position_ids
  ids_t_flat = input_ids.astype(jnp.int32).T.reshape(ROWS)
  pos_flat = jnp.concatenate(
      [pos_table.reshape(MAX_POS * HIDDEN),
       jnp.zeros((NPOS * HIDDEN,), jnp.float32)])
  out = _run(ids_t_flat, token_table, pos_flat)
  return out.reshape(SEQ, BATCH, HIDDEN).transpose(1, 0, 2)
